# baseline (speedup 1.0000x reference)
import jax
import jax.numpy as jnp
from jax import lax
from jax.experimental import pallas as pl
from jax.experimental.pallas import tpu as pltpu
from jax.experimental.pallas import tpu_sc as plsc

BATCH = 1024
SEQ = 200
D = 64
SCALE = 8.0

NC, NS, L = 2, 16, 16
NW = NC * NS
ROWS = BATCH * SEQ
RPW = ROWS // NW
K = 32
CHUNKS = RPW // K
NBUF = 2
GROUPS = CHUNKS // NBUF


def _body(tok_hbm, pe_hbm, table_hbm, out_hbm,
          tok_v, pe_v, tok_s,
          w0, w1, ob0, ob1, sg0, sg1, so0, so1):
  wins = (w0, w1)
  obs = (ob0, ob1)
  sg = (sg0, sg1)
  so = (so0, so1)

  wid = lax.axis_index("s") * NC + lax.axis_index("c")
  row_base = wid * RPW

  pltpu.sync_copy(tok_hbm.at[pl.ds(row_base, RPW)], tok_v)
  pltpu.sync_copy(pe_hbm, pe_v)

  iota = lax.iota(jnp.int32, L)

  def extract_and_fire(c, b):
    coff = c * K
    for g in range(K // L):
      t16 = tok_v[pl.ds(coff + g * L, L)]
      for r in range(L):
        t = lax.reduce_max(jnp.where(iota == r, t16, 0), (0,))
        tok_s[b * K + g * L + r] = t
        pltpu.async_copy(table_hbm.at[pl.ds((t >> 3) * 8, 8)],
                         wins[b].at[g * L + r], sg[b])

  def drain_compute(c, b):
    coff = c * K
    gbase = row_base + coff

    @pl.loop(0, K)
    def _(r):
      t = tok_s[b * K + r]
      pltpu.make_async_copy(table_hbm.at[pl.ds((t >> 3) * 8, 8)],
                            wins[b].at[r], sg[b]).wait()
      row = t & 7
      prow = lax.rem(gbase + r, SEQ)
      for j in range(D // L):
        x = wins[b][r, row, pl.ds(j * L, L)] * SCALE
        obs[b][pl.ds(r * D + j * L, L)] = x + pe_v[prow, pl.ds(j * L, L)]

  def fire_out(c, b):
    pltpu.async_copy(obs[b],
                     out_hbm.at[pl.ds((row_base + c * K) * D, K * D)], so[b])

  def wait_out(c, b):
    pltpu.make_async_copy(
        obs[b], out_hbm.at[pl.ds((row_base + c * K) * D, K * D)],
        so[b]).wait()

  for b in range(NBUF):
    extract_and_fire(b, b)

  for b in range(NBUF):
    drain_compute(b, b)
    fire_out(b, b)
    extract_and_fire(NBUF + b, b)

  @pl.loop(1, GROUPS - 1)
  def _(g):
    for b in range(NBUF):
      c = g * NBUF + b
      drain_compute(c, b)
      wait_out(c - NBUF, b)
      fire_out(c, b)
      extract_and_fire(c + NBUF, b)

  for b in range(NBUF):
    c = (GROUPS - 1) * NBUF + b
    drain_compute(c, b)
    wait_out(c - NBUF, b)
    fire_out(c, b)

  for b in range(NBUF):
    wait_out((GROUPS - 1) * NBUF + b, b)


@jax.jit
def _embed(tok, pe200, table):
  mesh = plsc.VectorSubcoreMesh(core_axis_name="c", subcore_axis_name="s")
  f = pl.kernel(
      _body,
      out_type=jax.ShapeDtypeStruct((ROWS * D,), jnp.float32),
      mesh=mesh,
      scratch_types=(
          [pltpu.VMEM((RPW,), jnp.int32),
           pltpu.VMEM((SEQ, D), jnp.float32),
           pltpu.SMEM((NBUF * K,), jnp.int32)]
          + [pltpu.VMEM((K, 8, D), jnp.float32) for _ in range(NBUF)]
          + [pltpu.VMEM((K * D,), jnp.float32) for _ in range(NBUF)]
          + [pltpu.SemaphoreType.DMA for _ in range(2 * NBUF)]
      ),
      compiler_params=pltpu.CompilerParams(needs_layout_passes=False),
  )
  return f(tok, pe200, table)


def kernel(tokens, table, pe):
  tok = tokens.astype(jnp.int32).reshape(ROWS)
  out = _embed(tok, pe[:SEQ], table)
  return out.reshape(BATCH, SEQ, D)
